# Initial kernel scaffold; baseline (speedup 1.0000x reference)
import functools

import jax
import jax.numpy as jnp
from jax import lax
from jax.experimental import pallas as pl
from jax.experimental.pallas import tpu as pltpu
from jax.experimental.pallas import tpu_sc as plsc

_info = plsc.get_sparse_core_info()
_NC = _info.num_cores
_NS = _info.num_subcores
_NW = _NC * _NS


@functools.lru_cache(maxsize=None)
def _make_gather(vocab, dim, n, chunk):
  n_per_w = n // _NW
  n_chunks = n_per_w // chunk
  mesh = plsc.VectorSubcoreMesh(core_axis_name="c", subcore_axis_name="s")

  @functools.partial(
      pl.kernel,
      mesh=mesh,
      out_type=jax.ShapeDtypeStruct((n, dim), jnp.float32),
      scratch_types=[
          pltpu.VMEM((n_per_w,), jnp.int32),
          pltpu.VMEM((chunk, dim), jnp.float32),
          pltpu.SemaphoreType.DMA,
      ],
  )
  def gather_kernel(table_hbm, idx_hbm, out_hbm, idx_v, rows_v, sem):
    wid = lax.axis_index("s") * _NC + lax.axis_index("c")
    base = wid * n_per_w
    pltpu.sync_copy(idx_hbm.at[pl.ds(base, n_per_w)], idx_v)

    def body(c, carry):
      off = c * chunk
      pltpu.async_copy(
          table_hbm.at[idx_v.at[pl.ds(off, chunk)]], rows_v, sem
      ).wait()
      pltpu.sync_copy(rows_v, out_hbm.at[pl.ds(base + off, chunk)])
      return carry

    lax.fori_loop(0, n_chunks, body, 0)

  return gather_kernel


def kernel(inp, table):
  b, s = inp.shape
  vocab, dim = table.shape
  n = b * s
  flat = inp.reshape(n).astype(jnp.int32)
  chunk = 1024
  pad = (-n) % (_NW * chunk)
  if pad:
    flat = jnp.concatenate([flat, jnp.zeros((pad,), jnp.int32)])
  out = _make_gather(vocab, dim, n + pad, chunk)(table, flat)
  if pad:
    out = out[:n]
  return out.reshape(b, s, dim)


# speedup vs baseline: 1.4782x; 1.4782x over previous
import functools

import jax
import jax.numpy as jnp
from jax import lax
from jax.experimental import pallas as pl
from jax.experimental.pallas import tpu as pltpu
from jax.experimental.pallas import tpu_sc as plsc

_info = plsc.get_sparse_core_info()
_NC = _info.num_cores
_NS = _info.num_subcores
_NW = _NC * _NS


@functools.lru_cache(maxsize=None)
def _make_gather(vocab, dim, n, chunk):
  n_per_w = n // _NW
  n_chunks = n_per_w // chunk
  mesh = plsc.VectorSubcoreMesh(core_axis_name="c", subcore_axis_name="s")

  @functools.partial(
      pl.kernel,
      mesh=mesh,
      out_type=jax.ShapeDtypeStruct((n, dim), jnp.float32),
      compiler_params=pltpu.CompilerParams(use_tc_tiling_on_sc=False),
      scratch_types=[
          pltpu.VMEM((n_per_w,), jnp.int32),
          pltpu.VMEM((chunk, dim), jnp.float32),
          pltpu.SemaphoreType.DMA,
      ],
  )
  def gather_kernel(table_hbm, idx_hbm, out_hbm, idx_v, rows_v, sem):
    wid = lax.axis_index("s") * _NC + lax.axis_index("c")
    base = wid * n_per_w
    pltpu.sync_copy(idx_hbm.at[pl.ds(base, n_per_w)], idx_v)

    def body(c, carry):
      off = c * chunk
      pltpu.async_copy(
          table_hbm.at[idx_v.at[pl.ds(off, chunk)]], rows_v, sem
      ).wait()
      pltpu.sync_copy(rows_v, out_hbm.at[pl.ds(base + off, chunk)])
      return carry

    lax.fori_loop(0, n_chunks, body, 0)

  return gather_kernel


def kernel(inp, table):
  b, s = inp.shape
  vocab, dim = table.shape
  n = b * s
  flat = inp.reshape(n).astype(jnp.int32)
  chunk = 1024
  pad = (-n) % (_NW * chunk)
  if pad:
    flat = jnp.concatenate([flat, jnp.zeros((pad,), jnp.int32)])
  out = _make_gather(vocab, dim, n + pad, chunk)(table, flat)
  if pad:
    out = out[:n]
  return out.reshape(b, s, dim)


# speedup vs baseline: 1.4964x; 1.0124x over previous
import functools

import jax
import jax.numpy as jnp
from jax import lax
from jax.experimental import pallas as pl
from jax.experimental.pallas import tpu as pltpu
from jax.experimental.pallas import tpu_sc as plsc

_info = plsc.get_sparse_core_info()
_NC = _info.num_cores
_NS = _info.num_subcores
_NW = _NC * _NS


@functools.lru_cache(maxsize=None)
def _make_gather(vocab, dim, n, chunk, nbuf):
  n_per_w = n // _NW
  n_chunks = n_per_w // chunk
  n_groups = n_chunks // nbuf
  mesh = plsc.VectorSubcoreMesh(core_axis_name="c", subcore_axis_name="s")

  @functools.partial(
      pl.kernel,
      mesh=mesh,
      out_type=jax.ShapeDtypeStruct((n, dim), jnp.float32),
      compiler_params=pltpu.CompilerParams(use_tc_tiling_on_sc=False),
      scratch_types=[
          pltpu.VMEM((n_per_w,), jnp.int32),
          pltpu.VMEM((nbuf, chunk, dim), jnp.float32),
      ]
      + [pltpu.SemaphoreType.DMA] * (2 * nbuf),
  )
  def gather_kernel(table_hbm, idx_hbm, out_hbm, idx_v, rows_v, *sems):
    gsem = sems[:nbuf]
    osem = sems[nbuf:]
    wid = lax.axis_index("s") * _NC + lax.axis_index("c")
    base = wid * n_per_w
    pltpu.sync_copy(idx_hbm.at[pl.ds(base, n_per_w)], idx_v)

    def start_gather(b, c):
      pltpu.async_copy(
          table_hbm.at[idx_v.at[pl.ds(c * chunk, chunk)]],
          rows_v.at[b],
          gsem[b],
      )

    def wait_gather(b):
      pltpu.make_async_copy(
          table_hbm.at[pl.ds(0, chunk)], rows_v.at[b], gsem[b]
      ).wait()

    def start_out(b, c):
      pltpu.async_copy(
          rows_v.at[b], out_hbm.at[pl.ds(base + c * chunk, chunk)], osem[b]
      )

    def wait_out(b):
      pltpu.make_async_copy(
          out_hbm.at[pl.ds(base, chunk)], rows_v.at[b], osem[b]
      ).wait()

    for b in range(nbuf):
      start_gather(b, b)

    def group_body(g, carry):
      c0 = g * nbuf
      for b in range(nbuf):
        wait_gather(b)
        start_out(b, c0 + b)
      for b in range(nbuf):
        wait_out(b)
        start_gather(b, c0 + nbuf + b)
      return carry

    lax.fori_loop(0, n_groups - 1, group_body, 0)

    c0 = (n_groups - 1) * nbuf
    for b in range(nbuf):
      wait_gather(b)
      start_out(b, c0 + b)
    for b in range(nbuf):
      wait_out(b)

  return gather_kernel


def kernel(inp, table):
  b, s = inp.shape
  vocab, dim = table.shape
  n = b * s
  flat = inp.reshape(n).astype(jnp.int32)
  chunk, nbuf = 640, 5
  pad = (-n) % (_NW * chunk * nbuf)
  if pad:
    flat = jnp.concatenate([flat, jnp.zeros((pad,), jnp.int32)])
  out = _make_gather(vocab, dim, n + pad, chunk, nbuf)(table, flat)
  if pad:
    out = out[:n]
  return out.reshape(b, s, dim)
